# Initial kernel scaffold; baseline (speedup 1.0000x reference)
import jax
import jax.numpy as jnp
from jax.experimental import pallas as pl


def kernel(x, token_types, type_emb, W_in, b_in, edge_emb, Wq, Wk, Wv, We, Wo, ln_s, ln_b, Wg, bg, Wc, bc, lno_s, lno_b, W_out, b_out):
    raise NotImplementedError("write your pallas kernel here")


# speedup vs baseline: 204.2980x; 204.2980x over previous
import functools

import jax
import jax.numpy as jnp
import numpy as np
from jax.experimental import pallas as pl
from jax.experimental.pallas import tpu as pltpu

B, S, DIN = 2, 2048, 128
D, EDIM, L, H = 256, 128, 4, 4
DH = D // H
N = B * S
OFFS = (-3, -2, -1, 1, 2, 3)


def _layernorm(x, s, b, eps=1e-5):
    m = jnp.mean(x, axis=-1, keepdims=True)
    v = jnp.mean((x - m) ** 2, axis=-1, keepdims=True)
    return (x - m) / jnp.sqrt(v + eps) * s + b


def _fwd(x_ref, oh_ref, te_ref, Win_ref, bin_ref, erow_ref,
         Wq_ref, Wk_ref, Wv_ref, We_ref, Wo_ref, lns_ref, lnb_ref,
         Wg_ref, bg_ref, Wc_ref, bc_ref, lnos_ref, lnob_ref,
         Wout_ref, bout_ref, out_ref):
    f32 = jnp.float32
    dot = functools.partial(jnp.dot, preferred_element_type=f32)

    h = dot(x_ref[...], Win_ref[...]) + bin_ref[...]
    h = h + dot(oh_ref[...], te_ref[...])

    pos = jax.lax.broadcasted_iota(jnp.int32, (N, 1), 0) % S

    di = jax.lax.broadcasted_iota(jnp.int32, (D, H), 0)
    hi = jax.lax.broadcasted_iota(jnp.int32, (D, H), 1)
    seg = (di // DH == hi).astype(f32)
    inv_sqrt = f32(1.0 / np.sqrt(DH))

    for l in range(L):
        q = dot(h, Wq_ref[l])
        k = dot(h, Wk_ref[l])
        v = dot(h, Wv_ref[l])
        e = dot(erow_ref[...], We_ref[l])
        ke = k + e
        ve = v + e

        scs = []
        for o in OFFS:
            ks = jnp.roll(ke, -o, axis=0)
            s = dot(q * ks, seg) * inv_sqrt
            valid = (pos + o >= 0) & (pos + o < S)
            scs.append(jnp.where(valid, s, f32(-1e9)))

        mx = scs[0]
        for s in scs[1:]:
            mx = jnp.maximum(mx, s)
        exs = [jnp.exp(s - mx) for s in scs]
        den = exs[0]
        for ex in exs[1:]:
            den = den + ex

        agg = jnp.zeros_like(h)
        for o, ex in zip(OFFS, exs):
            valid = ((pos + o >= 0) & (pos + o < S)).astype(f32)
            al = ex / (den + 1e-9) * valid
            ale = dot(al, seg.T)
            agg = agg + ale * jnp.roll(ve, -o, axis=0)

        h = _layernorm(h + dot(agg, Wo_ref[l]),
                       lns_ref[l:l + 1], lnb_ref[l:l + 1])

    gate = jax.nn.sigmoid(dot(h, Wg_ref[...]) + bg_ref[...])
    c = jnp.tanh(dot(h, Wc_ref[...]) + bc_ref[...])
    h = gate * h + (1.0 - gate) * c
    h = _layernorm(h, lnos_ref[...], lnob_ref[...])
    out_ref[...] = dot(h, Wout_ref[...]) + bout_ref[...]


@jax.jit
def kernel(x, token_types, type_emb, W_in, b_in, edge_emb, Wq, Wk, Wv, We, Wo,
           ln_s, ln_b, Wg, bg, Wc, bc, lno_s, lno_b, W_out, b_out):
    x2 = x.reshape(N, DIN)
    oh = jax.nn.one_hot(token_types.reshape(-1), 8, dtype=jnp.float32)
    te = jnp.concatenate([type_emb, jnp.zeros((2, D), jnp.float32)], axis=0)
    erow = edge_emb[5:6]

    out = pl.pallas_call(
        _fwd,
        out_shape=jax.ShapeDtypeStruct((N, DIN), jnp.float32),
        compiler_params=pltpu.CompilerParams(
            vmem_limit_bytes=120 * 1024 * 1024),
    )(x2, oh, te, W_in, b_in.reshape(1, D), erow,
      Wq, Wk, Wv, We, Wo, ln_s, ln_b,
      Wg, bg.reshape(1, D), Wc, bc.reshape(1, D),
      lno_s.reshape(1, D), lno_b.reshape(1, D),
      W_out, b_out.reshape(1, DIN))
    return out.reshape(B, S, DIN)


# speedup vs baseline: 209.7610x; 1.0267x over previous
import functools

import jax
import jax.numpy as jnp
import numpy as np
from jax.experimental import pallas as pl
from jax.experimental.pallas import tpu as pltpu

B, S, DIN = 2, 2048, 128
D, EDIM, L, H = 256, 128, 4, 4
DH = D // H
N = B * S
OFFS = (-3, -2, -1, 1, 2, 3)


def _layernorm(x, s, b, eps=1e-5):
    m = jnp.mean(x, axis=-1, keepdims=True)
    v = jnp.mean((x - m) ** 2, axis=-1, keepdims=True)
    return (x - m) / jnp.sqrt(v + eps) * s + b


def _fwd(x_ref, oh_ref, te_ref, Win_ref, bin_ref, erow_ref,
         Wq_ref, Wk_ref, Wv_ref, We_ref, Wo_ref, lns_ref, lnb_ref,
         Wg_ref, bg_ref, Wc_ref, bc_ref, lnos_ref, lnob_ref,
         Wout_ref, bout_ref, out_ref):
    f32 = jnp.float32
    dot = functools.partial(jnp.dot, preferred_element_type=f32)

    h = dot(x_ref[...], Win_ref[...]) + bin_ref[...]
    h = h + dot(oh_ref[...], te_ref[...])

    pos = jax.lax.broadcasted_iota(jnp.int32, (S, 1), 0)

    di = jax.lax.broadcasted_iota(jnp.int32, (D, H), 0)
    hi = jax.lax.broadcasted_iota(jnp.int32, (D, H), 1)
    seg = (di // DH == hi).astype(f32)
    inv_sqrt = f32(1.0 / np.sqrt(DH))

    for l in range(L):
        q = dot(h, Wq_ref[l])
        k = dot(h, Wk_ref[l])
        v = dot(h, Wv_ref[l])
        e = dot(erow_ref[...], We_ref[l])
        ke = k + e
        ve = v + e

        scs = []
        for o in OFFS:
            ks = jnp.roll(ke, -o, axis=0)
            s = dot(q * ks, seg) * inv_sqrt
            valid = (pos + o >= 0) & (pos + o < S)
            scs.append(jnp.where(valid, s, f32(-1e9)))

        mx = scs[0]
        for s in scs[1:]:
            mx = jnp.maximum(mx, s)
        exs = [jnp.exp(s - mx) for s in scs]
        den = exs[0]
        for ex in exs[1:]:
            den = den + ex

        agg = jnp.zeros_like(h)
        for o, ex in zip(OFFS, exs):
            valid = ((pos + o >= 0) & (pos + o < S)).astype(f32)
            al = ex / (den + 1e-9) * valid
            ale = dot(al, seg.T)
            agg = agg + ale * jnp.roll(ve, -o, axis=0)

        h = _layernorm(h + dot(agg, Wo_ref[l]),
                       lns_ref[l:l + 1], lnb_ref[l:l + 1])

    gate = jax.nn.sigmoid(dot(h, Wg_ref[...]) + bg_ref[...])
    c = jnp.tanh(dot(h, Wc_ref[...]) + bc_ref[...])
    h = gate * h + (1.0 - gate) * c
    h = _layernorm(h, lnos_ref[...], lnob_ref[...])
    out_ref[...] = dot(h, Wout_ref[...]) + bout_ref[...]


@jax.jit
def kernel(x, token_types, type_emb, W_in, b_in, edge_emb, Wq, Wk, Wv, We, Wo,
           ln_s, ln_b, Wg, bg, Wc, bc, lno_s, lno_b, W_out, b_out):
    x2 = x.reshape(N, DIN)
    oh = jax.nn.one_hot(token_types.reshape(-1), 8, dtype=jnp.float32)
    te = jnp.concatenate([type_emb, jnp.zeros((2, D), jnp.float32)], axis=0)
    erow = edge_emb[5:6]

    def full(a):
        return pl.BlockSpec(a.shape, lambda i: tuple(0 for _ in a.shape))

    weights = (W_in, b_in.reshape(1, D), erow,
               Wq, Wk, Wv, We, Wo, ln_s, ln_b,
               Wg, bg.reshape(1, D), Wc, bc.reshape(1, D),
               lno_s.reshape(1, D), lno_b.reshape(1, D),
               W_out, b_out.reshape(1, DIN))

    out = pl.pallas_call(
        _fwd,
        grid=(B,),
        in_specs=[pl.BlockSpec((S, DIN), lambda i: (i, 0)),
                  pl.BlockSpec((S, 8), lambda i: (i, 0)),
                  full(te)] + [full(w) for w in weights],
        out_specs=pl.BlockSpec((S, DIN), lambda i: (i, 0)),
        out_shape=jax.ShapeDtypeStruct((N, DIN), jnp.float32),
        compiler_params=pltpu.CompilerParams(
            dimension_semantics=("parallel",),
            vmem_limit_bytes=120 * 1024 * 1024),
    )(x2, oh, te, *weights)
    return out.reshape(B, S, DIN)


# speedup vs baseline: 210.1220x; 1.0017x over previous
import functools

import jax
import jax.numpy as jnp
import numpy as np
from jax.experimental import pallas as pl
from jax.experimental.pallas import tpu as pltpu

B, S, DIN = 2, 2048, 128
D, EDIM, L, H = 256, 128, 4, 4
DH = D // H
N = B * S
OFFS = (-3, -2, -1, 1, 2, 3)


def _layernorm(x, s, b, eps=1e-5):
    m = jnp.mean(x, axis=-1, keepdims=True)
    v = jnp.mean((x - m) ** 2, axis=-1, keepdims=True)
    return (x - m) / jnp.sqrt(v + eps) * s + b


def _fwd(x_ref, oh_ref, te_ref, Win_ref, bin_ref, erow_ref,
         Wq_ref, Wk_ref, Wv_ref, We_ref, Wo_ref, lns_ref, lnb_ref,
         Wg_ref, bg_ref, Wc_ref, bc_ref, lnos_ref, lnob_ref,
         Wout_ref, bout_ref, out_ref):
    f32 = jnp.float32
    dot = functools.partial(jnp.dot, preferred_element_type=f32)

    def bdot(a, b):
        return jnp.dot(a.astype(jnp.bfloat16), b.astype(jnp.bfloat16),
                       preferred_element_type=f32)

    h = bdot(x_ref[...], Win_ref[...]) + bin_ref[...]
    h = h + dot(oh_ref[...], te_ref[...])

    pos = jax.lax.broadcasted_iota(jnp.int32, (S, 1), 0)

    di = jax.lax.broadcasted_iota(jnp.int32, (D, H), 0)
    hi = jax.lax.broadcasted_iota(jnp.int32, (D, H), 1)
    seg = (di // DH == hi).astype(f32)
    inv_sqrt = f32(1.0 / np.sqrt(DH))

    for l in range(L):
        q = bdot(h, Wq_ref[l])
        k = bdot(h, Wk_ref[l])
        v = bdot(h, Wv_ref[l])
        e = dot(erow_ref[...], We_ref[l])
        ke = k + e
        ve = v + e

        scs = []
        for o in OFFS:
            ks = jnp.roll(ke, -o, axis=0)
            s = dot(q * ks, seg) * inv_sqrt
            valid = (pos + o >= 0) & (pos + o < S)
            scs.append(jnp.where(valid, s, f32(-1e9)))

        mx = scs[0]
        for s in scs[1:]:
            mx = jnp.maximum(mx, s)
        exs = [jnp.exp(s - mx) for s in scs]
        den = exs[0]
        for ex in exs[1:]:
            den = den + ex

        agg = jnp.zeros_like(h)
        for o, ex in zip(OFFS, exs):
            valid = ((pos + o >= 0) & (pos + o < S)).astype(f32)
            al = ex / (den + 1e-9) * valid
            ale = dot(al, seg.T)
            agg = agg + ale * jnp.roll(ve, -o, axis=0)

        h = _layernorm(h + bdot(agg, Wo_ref[l]),
                       lns_ref[l:l + 1], lnb_ref[l:l + 1])

    gate = jax.nn.sigmoid(bdot(h, Wg_ref[...]) + bg_ref[...])
    c = jnp.tanh(bdot(h, Wc_ref[...]) + bc_ref[...])
    h = gate * h + (1.0 - gate) * c
    h = _layernorm(h, lnos_ref[...], lnob_ref[...])
    out_ref[...] = bdot(h, Wout_ref[...]) + bout_ref[...]


@jax.jit
def kernel(x, token_types, type_emb, W_in, b_in, edge_emb, Wq, Wk, Wv, We, Wo,
           ln_s, ln_b, Wg, bg, Wc, bc, lno_s, lno_b, W_out, b_out):
    x2 = x.reshape(N, DIN)
    oh = jax.nn.one_hot(token_types.reshape(-1), 8, dtype=jnp.float32)
    te = jnp.concatenate([type_emb, jnp.zeros((2, D), jnp.float32)], axis=0)
    erow = edge_emb[5:6]

    def full(a):
        return pl.BlockSpec(a.shape, lambda i: tuple(0 for _ in a.shape))

    weights = (W_in, b_in.reshape(1, D), erow,
               Wq, Wk, Wv, We, Wo, ln_s, ln_b,
               Wg, bg.reshape(1, D), Wc, bc.reshape(1, D),
               lno_s.reshape(1, D), lno_b.reshape(1, D),
               W_out, b_out.reshape(1, DIN))

    out = pl.pallas_call(
        _fwd,
        grid=(B,),
        in_specs=[pl.BlockSpec((S, DIN), lambda i: (i, 0)),
                  pl.BlockSpec((S, 8), lambda i: (i, 0)),
                  full(te)] + [full(w) for w in weights],
        out_specs=pl.BlockSpec((S, DIN), lambda i: (i, 0)),
        out_shape=jax.ShapeDtypeStruct((N, DIN), jnp.float32),
        compiler_params=pltpu.CompilerParams(
            dimension_semantics=("parallel",),
            vmem_limit_bytes=120 * 1024 * 1024),
    )(x2, oh, te, *weights)
    return out.reshape(B, S, DIN)


# speedup vs baseline: 217.9190x; 1.0371x over previous
import functools

import jax
import jax.numpy as jnp
import numpy as np
from jax.experimental import pallas as pl
from jax.experimental.pallas import tpu as pltpu

B, S, DIN = 2, 2048, 128
D, EDIM, L, H = 256, 128, 4, 4
DH = D // H
N = B * S
OFFS = (-3, -2, -1, 1, 2, 3)


def _layernorm(x, s, b, eps=1e-5):
    m = jnp.mean(x, axis=-1, keepdims=True)
    v = jnp.mean((x - m) ** 2, axis=-1, keepdims=True)
    return (x - m) / jnp.sqrt(v + eps) * s + b


def _fwd(x_ref, oh_ref, te_ref, Win_ref, bin_ref, erow_ref,
         Wq_ref, Wk_ref, Wv_ref, We_ref, Wo_ref, lns_ref, lnb_ref,
         Wg_ref, bg_ref, Wc_ref, bc_ref, lnos_ref, lnob_ref,
         Wout_ref, bout_ref, out_ref):
    f32 = jnp.float32
    dot = functools.partial(jnp.dot, preferred_element_type=f32)

    def bdot(a, b):
        return jnp.dot(a.astype(jnp.bfloat16), b.astype(jnp.bfloat16),
                       preferred_element_type=f32)

    h = bdot(x_ref[...], Win_ref[...]) + bin_ref[...]
    h = h + dot(oh_ref[...], te_ref[...])

    pos = jax.lax.broadcasted_iota(jnp.int32, (S, 1), 0)

    di = jax.lax.broadcasted_iota(jnp.int32, (D, H), 0)
    hi = jax.lax.broadcasted_iota(jnp.int32, (D, H), 1)
    seg = (di // DH == hi).astype(f32)
    seg16 = seg.astype(jnp.bfloat16)
    inv_sqrt = f32(1.0 / np.sqrt(DH))

    bf16 = jnp.bfloat16
    for l in range(L):
        q = bdot(h, Wq_ref[l]).astype(bf16)
        k = bdot(h, Wk_ref[l])
        v = bdot(h, Wv_ref[l])
        e = dot(erow_ref[...], We_ref[l])
        kv = jnp.concatenate([(k + e).astype(bf16), v.astype(bf16)], axis=1)

        shifted, scs = [], []
        for o in OFFS:
            kvs = jnp.roll(kv, -o, axis=0)
            shifted.append(kvs)
            s = dot(q * kvs[:, :D], seg16) * inv_sqrt
            valid = (pos + o >= 0) & (pos + o < S)
            scs.append(jnp.where(valid, s, f32(-1e9)))

        mx = scs[0]
        for s in scs[1:]:
            mx = jnp.maximum(mx, s)
        exs = [jnp.exp(s - mx) for s in scs]
        den = exs[0]
        for ex in exs[1:]:
            den = den + ex

        agg = jnp.zeros_like(h)
        for o, ex, kvs in zip(OFFS, exs, shifted):
            valid = ((pos + o >= 0) & (pos + o < S)).astype(f32)
            al = ex / (den + 1e-9) * valid
            ale = dot(al, seg.T)
            agg = agg + ale * kvs[:, D:]
        agg = agg + e

        h = _layernorm(h + bdot(agg, Wo_ref[l]),
                       lns_ref[l:l + 1], lnb_ref[l:l + 1])

    gate = jax.nn.sigmoid(bdot(h, Wg_ref[...]) + bg_ref[...])
    c = jnp.tanh(bdot(h, Wc_ref[...]) + bc_ref[...])
    h = gate * h + (1.0 - gate) * c
    h = _layernorm(h, lnos_ref[...], lnob_ref[...])
    out_ref[...] = bdot(h, Wout_ref[...]) + bout_ref[...]


@jax.jit
def kernel(x, token_types, type_emb, W_in, b_in, edge_emb, Wq, Wk, Wv, We, Wo,
           ln_s, ln_b, Wg, bg, Wc, bc, lno_s, lno_b, W_out, b_out):
    x2 = x.reshape(N, DIN)
    oh = jax.nn.one_hot(token_types.reshape(-1), 8, dtype=jnp.float32)
    te = jnp.concatenate([type_emb, jnp.zeros((2, D), jnp.float32)], axis=0)
    erow = edge_emb[5:6]

    def full(a):
        return pl.BlockSpec(a.shape, lambda i: tuple(0 for _ in a.shape))

    weights = (W_in, b_in.reshape(1, D), erow,
               Wq, Wk, Wv, We, Wo, ln_s, ln_b,
               Wg, bg.reshape(1, D), Wc, bc.reshape(1, D),
               lno_s.reshape(1, D), lno_b.reshape(1, D),
               W_out, b_out.reshape(1, DIN))

    out = pl.pallas_call(
        _fwd,
        grid=(B,),
        in_specs=[pl.BlockSpec((S, DIN), lambda i: (i, 0)),
                  pl.BlockSpec((S, 8), lambda i: (i, 0)),
                  full(te)] + [full(w) for w in weights],
        out_specs=pl.BlockSpec((S, DIN), lambda i: (i, 0)),
        out_shape=jax.ShapeDtypeStruct((N, DIN), jnp.float32),
        compiler_params=pltpu.CompilerParams(
            dimension_semantics=("parallel",),
            vmem_limit_bytes=120 * 1024 * 1024),
    )(x2, oh, te, *weights)
    return out.reshape(B, S, DIN)
